# baseline (speedup 1.0000x reference)
import jax
import jax.numpy as jnp
from jax.experimental import pallas as pl

_N_GRID = 128
_R_S = 48
_R_C = 144
_P = 27
_CH = 128
_SIGMA_BIAS = -5.0
_BLK = 1024


def _leaky(x):
    return jnp.where(x >= 0, x, 0.01 * x)


def _sigmoid(x):
    z = jnp.exp(-jnp.abs(x))
    return jnp.where(x >= 0, 1.0 / (1.0 + z), z / (1.0 + z))


def _softplus(x):
    return jnp.maximum(x, 0.0) + jnp.log1p(jnp.exp(-jnp.abs(x)))


def _tc_body(xyz_ref, dirs_ref, voxel_ref, z_ref, tf1_ref, tmx_ref, ones_ref,
             bpa_ref, bpb_ref, e_ref, asin_ref, acos_ref, w2_ref, w3_ref,
             b1_ref, b2_ref, b3_ref, sig_ref, rgb_ref):
    blk = xyz_ref.shape[0]
    prod_f = None
    prod_m = None
    for a in range(3):
        xa = xyz_ref[:, a][:, None]
        vox = voxel_ref[a][None, :]
        cmp = (vox < xa).astype(jnp.float32)
        red = jnp.dot(cmp, z_ref[a], preferred_element_type=jnp.float32)
        vl = red[:, 0][:, None]
        vr = red[:, 1][:, None]
        lerp = (xa - vl) / (vr - vl + 1e-06)
        shl = jnp.concatenate([cmp[:, 1:], jnp.zeros((blk, 1), jnp.float32)],
                              axis=1)
        shr = jnp.concatenate([jnp.ones((blk, 1), jnp.float32), cmp[:, :-1]],
                              axis=1)
        wa = (cmp - shl) + lerp * (shr + shl - 2.0 * cmp)
        gf = jnp.dot(wa, tf1_ref[a], preferred_element_type=jnp.float32)
        gm = jnp.dot(wa, tmx_ref[a], preferred_element_type=jnp.float32)
        prod_f = gf if prod_f is None else prod_f * gf
        prod_m = gm if prod_m is None else prod_m * gm

    sig_raw = jnp.dot(prod_m, ones_ref[...],
                      preferred_element_type=jnp.float32)[:, 0] + _SIGMA_BIAS
    sig_ref[...] = _softplus(sig_raw)

    t = (jnp.dot(prod_f, bpa_ref[...], preferred_element_type=jnp.float32)
         + jnp.dot(prod_m, bpb_ref[...], preferred_element_type=jnp.float32)
         + jnp.dot(dirs_ref[...], e_ref[...],
                   preferred_element_type=jnp.float32))
    pre = (jnp.dot(jnp.sin(t), asin_ref[...],
                   preferred_element_type=jnp.float32)
           + jnp.dot(jnp.cos(t), acos_ref[...],
                     preferred_element_type=jnp.float32)
           + b1_ref[...][None, :])
    h1 = _leaky(pre)
    h2 = _leaky(jnp.dot(h1, w2_ref[...],
                        preferred_element_type=jnp.float32) + b2_ref[...][None, :])
    rgb_ref[...] = _sigmoid(
        jnp.dot(h2, w3_ref[...], preferred_element_type=jnp.float32)
        + b3_ref[...][None, :])


@jax.jit
def kernel(xyz, directions, voxel, sigma, feature, B, W1, b1, W2, b2, W3, b3):
    npts = xyz.shape[0]
    grid = npts // _BLK

    dl = jnp.concatenate([voxel[:, :1], voxel[:, 1:] - voxel[:, :-1]],
                         axis=1)[:, :, None]
    dr = jnp.concatenate([voxel[:, 1:] - voxel[:, :-1],
                          jnp.zeros((3, 1), jnp.float32)], axis=1)[:, :, None]
    dr = dr.at[:, 0, 0].add(voxel[:, 0])
    z = jnp.concatenate([dl, dr], axis=2)

    tf1 = jnp.transpose(feature[:, :_CH, :], (0, 2, 1))
    tmx = jnp.concatenate(
        [jnp.transpose(sigma, (0, 2, 1)),
         jnp.transpose(feature[:, _CH:, :], (0, 2, 1))], axis=2)
    ones48 = jnp.zeros((64, 1), jnp.float32).at[:_R_S].set(1.0)

    bp2 = (jnp.zeros((_R_C, _CH), jnp.float32)
           .at[:, :_P].set(B).at[:, _P:2 * _P].set(2.0 * B))
    bpa = bp2[:_CH]
    bpb = jnp.zeros((64, _CH), jnp.float32).at[_R_S:].set(bp2[_CH:])
    e = jnp.zeros((3, _CH), jnp.float32)
    for i in range(3):
        e = e.at[i, 54 + i].set(1.0).at[i, 57 + i].set(2.0)
    w1t = W1.T
    asin = (jnp.zeros((_CH, _CH), jnp.float32)
            .at[:_P].set(w1t[0:27]).at[_P:2 * _P].set(w1t[54:81])
            .at[54:57].set(w1t[108:111]).at[57:60].set(w1t[114:117]))
    acos = (jnp.zeros((_CH, _CH), jnp.float32)
            .at[:_P].set(w1t[27:54]).at[_P:2 * _P].set(w1t[81:108])
            .at[54:57].set(w1t[111:114]).at[57:60].set(w1t[117:120]))

    full = lambda *shape: pl.BlockSpec(shape, lambda i: (0,) * len(shape))
    sig, rgb = pl.pallas_call(
        _tc_body,
        grid=(grid,),
        in_specs=[
            pl.BlockSpec((_BLK, 3), lambda i: (i, 0)),
            pl.BlockSpec((_BLK, 3), lambda i: (i, 0)),
            full(3, _N_GRID),
            full(3, _N_GRID, 2),
            full(3, _N_GRID, _CH),
            full(3, _N_GRID, 64),
            full(64, 1),
            full(_CH, _CH),
            full(64, _CH),
            full(3, _CH),
            full(_CH, _CH),
            full(_CH, _CH),
            full(_CH, _CH),
            full(_CH, 3),
            full(_CH),
            full(_CH),
            full(3),
        ],
        out_specs=[
            pl.BlockSpec((_BLK,), lambda i: (i,)),
            pl.BlockSpec((_BLK, 3), lambda i: (i, 0)),
        ],
        out_shape=[
            jax.ShapeDtypeStruct((npts,), jnp.float32),
            jax.ShapeDtypeStruct((npts, 3), jnp.float32),
        ],
    )(xyz, directions, voxel, z, tf1, tmx, ones48, bpa, bpb,
      e, asin, acos, W2.T, W3.T, b1, b2, b3)
    return sig, rgb
